# Initial kernel scaffold; baseline (speedup 1.0000x reference)
import jax
import jax.numpy as jnp
from jax.experimental import pallas as pl


def kernel(x, edge_index, batch, W1, b1, W2, b2, W3, b3):
    raise NotImplementedError("write your pallas kernel here")


# speedup vs baseline: 7.0882x; 7.0882x over previous
import functools

import jax
import jax.numpy as jnp
from jax import lax
from jax.experimental import pallas as pl
from jax.experimental.pallas import tpu as pltpu
from jax.experimental.pallas import tpu_sc as plsc

N = 10000
E = 320000
D_FEAT = 128
D_HID = 128
N_CLASSES = 64

CHUNK = 128
N_CHUNKS = 2560
E_PAD = N_CHUNKS * CHUNK
CPT = N_CHUNKS // 32
IBLK = 16
ROW_SPLIT = 624
ACC_ROWS = 10016
DEG_SLOTS = 10016

_MESH = plsc.VectorSubcoreMesh(
    core_axis_name="c", subcore_axis_name="s", num_cores=2, num_subcores=16)


def _deg_body(dst_hbm, out_hbm, idx_v, hist):
    c = lax.axis_index("c")
    s = lax.axis_index("s")
    wid = c * 16 + s
    zero16 = jnp.zeros((16,), jnp.float32)

    def zstep(i, _):
        hist[pl.ds(16 * i, 16)] = zero16
        return 0

    lax.fori_loop(0, DEG_SLOTS // 16, zstep, 0)
    row0 = wid * CPT
    pltpu.sync_copy(dst_hbm.at[pl.ds(row0, CPT), :], idx_v)

    ones16 = jnp.ones((16,), jnp.float32)

    def step(g, _):
        for j in range(0, CHUNK, 16):
            plsc.addupdate_scatter(hist, [idx_v[g, pl.ds(j, 16)]], ones16)
        return 0

    lax.fori_loop(0, CPT, step, 0)
    pltpu.sync_copy(hist, out_hbm.at[wid])


_deg_kernel = functools.partial(
    pl.kernel,
    out_type=jax.ShapeDtypeStruct((32, DEG_SLOTS), jnp.float32),
    mesh=_MESH,
    compiler_params=pltpu.CompilerParams(needs_layout_passes=False),
    scratch_types=[
        pltpu.VMEM((CPT, CHUNK), jnp.int32),
        pltpu.VMEM((DEG_SLOTS,), jnp.float32),
    ],
)(_deg_body)


def _agg_body(src_hbm, dst_hbm, hp, out_hbm,
              src_idx, dst_idx, rows, acc, sem0, sem1):
    c = lax.axis_index("c")
    s = lax.axis_index("s")
    off = s * ROW_SPLIT
    pl.when(s < 15)(lambda: pltpu.sync_copy(
        hp.at[pl.ds(off, ROW_SPLIT), :], acc.at[pl.ds(off, ROW_SPLIT), :]))
    pl.when(s == 15)(lambda: pltpu.sync_copy(
        hp.at[pl.ds(15 * ROW_SPLIT, 640), :],
        acc.at[pl.ds(15 * ROW_SPLIT, 640), :]))
    row0 = c * (16 * CPT) + s * CPT
    plsc.subcore_barrier()

    def block(blk, _):
        pltpu.sync_copy(src_hbm.at[pl.ds(row0 + blk * IBLK, IBLK), :],
                        src_idx)
        pltpu.sync_copy(dst_hbm.at[pl.ds(row0 + blk * IBLK, IBLK), :],
                        dst_idx)

        def step(i, _):
            g = 2 * i
            cp0 = pltpu.async_copy(hp.at[src_idx.at[g]], rows.at[0], sem0)
            cp1 = pltpu.async_copy(hp.at[src_idx.at[g + 1]], rows.at[1],
                                   sem1)
            cp0.wait()
            pltpu.sync_copy(rows.at[0], acc.at[dst_idx.at[g]], add=True)
            cp1.wait()
            pltpu.sync_copy(rows.at[1], acc.at[dst_idx.at[g + 1]], add=True)
            return 0

        lax.fori_loop(0, IBLK // 2, step, 0)
        return 0

    lax.fori_loop(0, CPT // IBLK, block, 0)
    plsc.subcore_barrier()
    pl.when(s < 15)(lambda: pltpu.sync_copy(
        acc.at[pl.ds(off, ROW_SPLIT), :],
        out_hbm.at[c, pl.ds(off, ROW_SPLIT), :]))
    pl.when(s == 15)(lambda: pltpu.sync_copy(
        acc.at[pl.ds(15 * ROW_SPLIT, 640), :],
        out_hbm.at[c, pl.ds(15 * ROW_SPLIT, 640), :]))


_agg_kernel = functools.partial(
    pl.kernel,
    out_type=jax.ShapeDtypeStruct((2, N, D_HID), jnp.float32),
    mesh=_MESH,
    scratch_types=[
        pltpu.VMEM((IBLK, CHUNK), jnp.int32),
        pltpu.VMEM((IBLK, CHUNK), jnp.int32),
        pltpu.VMEM((2, CHUNK, D_HID), jnp.float32),
        pltpu.VMEM_SHARED((ACC_ROWS, D_HID), jnp.float32),
        pltpu.SemaphoreType.DMA,
        pltpu.SemaphoreType.DMA,
    ],
)(_agg_body)


_BLK = 400
_GRID = N // _BLK


def _tc_first_body(x_ref, deg_ref, w_ref, hp_ref, dinv_ref):
    deg = jnp.sum(deg_ref[...], axis=0) + 1.0
    dinv = lax.rsqrt(deg)
    h = jnp.dot(x_ref[...], w_ref[...], preferred_element_type=jnp.float32)
    hp_ref[...] = h * dinv
    dinv_ref[...] = dinv


def _tc_first(x, deg3, W1):
    return pl.pallas_call(
        _tc_first_body,
        grid=(_GRID,),
        in_specs=[
            pl.BlockSpec((_BLK, D_FEAT), lambda i: (i, 0)),
            pl.BlockSpec((32, _BLK, 1), lambda i: (0, i, 0)),
            pl.BlockSpec((D_FEAT, D_HID), lambda i: (0, 0)),
        ],
        out_specs=[
            pl.BlockSpec((_BLK, D_HID), lambda i: (i, 0)),
            pl.BlockSpec((_BLK, 1), lambda i: (i, 0)),
        ],
        out_shape=[
            jax.ShapeDtypeStruct((N, D_HID), jnp.float32),
            jax.ShapeDtypeStruct((N, 1), jnp.float32),
        ],
    )(x, deg3, W1)


def _tc_mid_body(a_ref, hp_ref, dinv_ref, b_ref, w_ref, out_ref):
    agg = a_ref[0] + a_ref[1] - hp_ref[...]
    dinv = dinv_ref[...]
    z = jnp.maximum(agg * dinv + b_ref[...], 0.0)
    h = jnp.dot(z, w_ref[...], preferred_element_type=jnp.float32)
    out_ref[...] = h * dinv


def _tc_mid(a, hp, dinv, b, W):
    return pl.pallas_call(
        _tc_mid_body,
        grid=(_GRID,),
        in_specs=[
            pl.BlockSpec((2, _BLK, D_HID), lambda i: (0, i, 0)),
            pl.BlockSpec((_BLK, D_HID), lambda i: (i, 0)),
            pl.BlockSpec((_BLK, 1), lambda i: (i, 0)),
            pl.BlockSpec((1, D_HID), lambda i: (0, 0)),
            pl.BlockSpec((D_HID, D_HID), lambda i: (0, 0)),
        ],
        out_specs=pl.BlockSpec((_BLK, D_HID), lambda i: (i, 0)),
        out_shape=jax.ShapeDtypeStruct((N, D_HID), jnp.float32),
    )(a, hp, dinv, b, W)


def _tc_scale_body(a_ref, hp_ref, dinv_ref, b_ref, out_ref):
    agg = a_ref[0] + a_ref[1] - hp_ref[...]
    dinv = dinv_ref[...]
    z = jnp.maximum(agg * dinv + b_ref[...], 0.0)
    out_ref[...] = z * dinv


def _tc_scale(a, hp, dinv, b):
    return pl.pallas_call(
        _tc_scale_body,
        grid=(_GRID,),
        in_specs=[
            pl.BlockSpec((2, _BLK, D_HID), lambda i: (0, i, 0)),
            pl.BlockSpec((_BLK, D_HID), lambda i: (i, 0)),
            pl.BlockSpec((_BLK, 1), lambda i: (i, 0)),
            pl.BlockSpec((1, D_HID), lambda i: (0, 0)),
        ],
        out_specs=pl.BlockSpec((_BLK, D_HID), lambda i: (i, 0)),
        out_shape=jax.ShapeDtypeStruct((N, D_HID), jnp.float32),
    )(a, hp, dinv, b)


def _tc_last_body(a_ref, hp_ref, dinv_ref, b_ref, w_ref, out_ref):
    agg = a_ref[0] + a_ref[1] - hp_ref[...]
    h = jnp.dot(agg, w_ref[...], preferred_element_type=jnp.float32)
    y = h * dinv_ref[...] + b_ref[...]
    m = jnp.max(y, axis=1, keepdims=True)
    lse = jnp.log(jnp.sum(jnp.exp(y - m), axis=1, keepdims=True))
    out_ref[...] = y - m - lse


def _tc_last(a, hp, dinv, b3, W3):
    return pl.pallas_call(
        _tc_last_body,
        grid=(_GRID,),
        in_specs=[
            pl.BlockSpec((2, _BLK, D_HID), lambda i: (0, i, 0)),
            pl.BlockSpec((_BLK, D_HID), lambda i: (i, 0)),
            pl.BlockSpec((_BLK, 1), lambda i: (i, 0)),
            pl.BlockSpec((1, N_CLASSES), lambda i: (0, 0)),
            pl.BlockSpec((D_HID, N_CLASSES), lambda i: (0, 0)),
        ],
        out_specs=pl.BlockSpec((_BLK, N_CLASSES), lambda i: (i, 0)),
        out_shape=jax.ShapeDtypeStruct((N, N_CLASSES), jnp.float32),
    )(a, hp, dinv, b3, W3)


@jax.jit
def kernel(x, edge_index, batch, W1, b1, W2, b2, W3, b3):
    src = edge_index[0]
    dst = edge_index[1]
    pad = E_PAD - E
    src_p = jnp.concatenate(
        [src, jnp.zeros((pad,), jnp.int32)]).reshape(N_CHUNKS, CHUNK)
    dst_p = jnp.concatenate(
        [dst, jnp.full((pad,), N, jnp.int32)]).reshape(N_CHUNKS, CHUNK)

    deg2 = _deg_kernel(dst_p)
    deg3 = deg2[:, :N, None]

    hp1, dinv = _tc_first(x, deg3, W1)
    a1 = _agg_kernel(src_p, dst_p, hp1)
    hp2 = _tc_mid(a1, hp1, dinv, b1.reshape(1, -1), W2)
    a2 = _agg_kernel(src_p, dst_p, hp2)
    hpz = _tc_scale(a2, hp2, dinv, b2.reshape(1, -1))
    a3 = _agg_kernel(src_p, dst_p, hpz)
    return _tc_last(a3, hpz, dinv, b3.reshape(1, -1), W3)


# speedup vs baseline: 7.7590x; 1.0946x over previous
import functools

import jax
import jax.numpy as jnp
from jax import lax
from jax.experimental import pallas as pl
from jax.experimental.pallas import tpu as pltpu
from jax.experimental.pallas import tpu_sc as plsc

N = 10000
E = 320000
D_FEAT = 128
D_HID = 128
N_CLASSES = 64

CHUNK = 128
N_CHUNKS = 2560
E_PAD = N_CHUNKS * CHUNK
CPT = N_CHUNKS // 32
IBLK = 16
ROW_SPLIT = 624
ACC_ROWS = 10016
DEG_SLOTS = 10016

_MESH = plsc.VectorSubcoreMesh(
    core_axis_name="c", subcore_axis_name="s", num_cores=2, num_subcores=16)


def _deg_body(dst_hbm, out_hbm, idx_v, hist):
    c = lax.axis_index("c")
    s = lax.axis_index("s")
    wid = c * 16 + s
    zero16 = jnp.zeros((16,), jnp.float32)

    def zstep(i, _):
        hist[pl.ds(16 * i, 16)] = zero16
        return 0

    lax.fori_loop(0, DEG_SLOTS // 16, zstep, 0)
    row0 = wid * CPT
    pltpu.sync_copy(dst_hbm.at[pl.ds(row0, CPT), :], idx_v)

    ones16 = jnp.ones((16,), jnp.float32)

    def step(g, _):
        for j in range(0, CHUNK, 16):
            plsc.addupdate_scatter(hist, [idx_v[g, pl.ds(j, 16)]], ones16)
        return 0

    lax.fori_loop(0, CPT, step, 0)
    pltpu.sync_copy(hist, out_hbm.at[wid])


_deg_kernel = functools.partial(
    pl.kernel,
    out_type=jax.ShapeDtypeStruct((32, DEG_SLOTS), jnp.float32),
    mesh=_MESH,
    compiler_params=pltpu.CompilerParams(needs_layout_passes=False),
    scratch_types=[
        pltpu.VMEM((CPT, CHUNK), jnp.int32),
        pltpu.VMEM((DEG_SLOTS,), jnp.float32),
    ],
)(_deg_body)


def _agg_body(src_hbm, dst_hbm, hp, out_hbm,
              src_idx, dst_idx, rows, acc, sem0, sem1):
    c = lax.axis_index("c")
    s = lax.axis_index("s")
    off = s * ROW_SPLIT
    pl.when(s < 15)(lambda: pltpu.sync_copy(
        hp.at[pl.ds(off, ROW_SPLIT), :], acc.at[pl.ds(off, ROW_SPLIT), :]))
    pl.when(s == 15)(lambda: pltpu.sync_copy(
        hp.at[pl.ds(15 * ROW_SPLIT, 640), :],
        acc.at[pl.ds(15 * ROW_SPLIT, 640), :]))
    row0 = c * (16 * CPT) + s * CPT
    plsc.subcore_barrier()

    def block(blk, _):
        pltpu.sync_copy(src_hbm.at[pl.ds(row0 + blk * IBLK, IBLK), :],
                        src_idx)
        pltpu.sync_copy(dst_hbm.at[pl.ds(row0 + blk * IBLK, IBLK), :],
                        dst_idx)

        def step(i, _):
            g = 2 * i
            cp0 = pltpu.async_copy(hp.at[src_idx.at[g]], rows.at[0], sem0)
            cp1 = pltpu.async_copy(hp.at[src_idx.at[g + 1]], rows.at[1],
                                   sem1)
            cp0.wait()
            pltpu.sync_copy(rows.at[0], acc.at[dst_idx.at[g]], add=True)
            cp1.wait()
            pltpu.sync_copy(rows.at[1], acc.at[dst_idx.at[g + 1]], add=True)
            return 0

        lax.fori_loop(0, IBLK // 2, step, 0)
        return 0

    lax.fori_loop(0, CPT // IBLK, block, 0)
    plsc.subcore_barrier()
    pl.when(s < 15)(lambda: pltpu.sync_copy(
        acc.at[pl.ds(off, ROW_SPLIT), :],
        out_hbm.at[c, pl.ds(off, ROW_SPLIT), :]))
    pl.when(s == 15)(lambda: pltpu.sync_copy(
        acc.at[pl.ds(15 * ROW_SPLIT, 640), :],
        out_hbm.at[c, pl.ds(15 * ROW_SPLIT, 640), :]))


_agg_kernel = functools.partial(
    pl.kernel,
    out_type=jax.ShapeDtypeStruct((2, N, D_HID), jnp.float32),
    mesh=_MESH,
    scratch_types=[
        pltpu.VMEM((IBLK, CHUNK), jnp.int32),
        pltpu.VMEM((IBLK, CHUNK), jnp.int32),
        pltpu.VMEM((2, CHUNK, D_HID), jnp.float32),
        pltpu.VMEM_SHARED((ACC_ROWS, D_HID), jnp.float32),
        pltpu.SemaphoreType.DMA,
        pltpu.SemaphoreType.DMA,
    ],
)(_agg_body)


_BLK = 400
_GRID = N // _BLK


def _tc_first_body(x_ref, deg_ref, w_ref, hp_ref, dinv_ref):
    deg = jnp.sum(deg_ref[...], axis=0) + 1.0
    dinv = lax.rsqrt(deg)
    h = jnp.dot(x_ref[...], w_ref[...], preferred_element_type=jnp.float32)
    hp_ref[...] = h * dinv
    dinv_ref[...] = dinv


def _tc_first(x, deg3, W1):
    return pl.pallas_call(
        _tc_first_body,
        grid=(_GRID,),
        in_specs=[
            pl.BlockSpec((_BLK, D_FEAT), lambda i: (i, 0)),
            pl.BlockSpec((32, _BLK, 1), lambda i: (0, i, 0)),
            pl.BlockSpec((D_FEAT, D_HID), lambda i: (0, 0)),
        ],
        out_specs=[
            pl.BlockSpec((_BLK, D_HID), lambda i: (i, 0)),
            pl.BlockSpec((_BLK, 1), lambda i: (i, 0)),
        ],
        out_shape=[
            jax.ShapeDtypeStruct((N, D_HID), jnp.float32),
            jax.ShapeDtypeStruct((N, 1), jnp.float32),
        ],
    )(x, deg3, W1)


def _tc_mid_body(a_ref, hp_ref, dinv_ref, b_ref, w_ref, out_ref):
    agg = a_ref[0] + a_ref[1] - hp_ref[...]
    dinv = dinv_ref[...]
    z = jnp.maximum(agg * dinv + b_ref[...], 0.0)
    h = jnp.dot(z, w_ref[...], preferred_element_type=jnp.float32)
    out_ref[...] = h * dinv


def _tc_mid(a, hp, dinv, b, W):
    return pl.pallas_call(
        _tc_mid_body,
        grid=(_GRID,),
        in_specs=[
            pl.BlockSpec((2, _BLK, D_HID), lambda i: (0, i, 0)),
            pl.BlockSpec((_BLK, D_HID), lambda i: (i, 0)),
            pl.BlockSpec((_BLK, 1), lambda i: (i, 0)),
            pl.BlockSpec((1, D_HID), lambda i: (0, 0)),
            pl.BlockSpec((D_HID, D_HID), lambda i: (0, 0)),
        ],
        out_specs=pl.BlockSpec((_BLK, D_HID), lambda i: (i, 0)),
        out_shape=jax.ShapeDtypeStruct((N, D_HID), jnp.float32),
    )(a, hp, dinv, b, W)


def _tc_scale_body(a_ref, hp_ref, dinv_ref, b_ref, out_ref):
    agg = a_ref[0] + a_ref[1] - hp_ref[...]
    dinv = dinv_ref[...]
    z = jnp.maximum(agg * dinv + b_ref[...], 0.0)
    out_ref[...] = z * dinv


def _tc_scale(a, hp, dinv, b):
    return pl.pallas_call(
        _tc_scale_body,
        grid=(_GRID,),
        in_specs=[
            pl.BlockSpec((2, _BLK, D_HID), lambda i: (0, i, 0)),
            pl.BlockSpec((_BLK, D_HID), lambda i: (i, 0)),
            pl.BlockSpec((_BLK, 1), lambda i: (i, 0)),
            pl.BlockSpec((1, D_HID), lambda i: (0, 0)),
        ],
        out_specs=pl.BlockSpec((_BLK, D_HID), lambda i: (i, 0)),
        out_shape=jax.ShapeDtypeStruct((N, D_HID), jnp.float32),
    )(a, hp, dinv, b)


def _tc_last_body(a_ref, hp_ref, dinv_ref, b_ref, w_ref, out_ref):
    agg = a_ref[0] + a_ref[1] - hp_ref[...]
    h = jnp.dot(agg, w_ref[...], preferred_element_type=jnp.float32)
    y = h * dinv_ref[...] + b_ref[...]
    m = jnp.max(y, axis=1, keepdims=True)
    lse = jnp.log(jnp.sum(jnp.exp(y - m), axis=1, keepdims=True))
    out_ref[...] = y - m - lse


def _tc_last(a, hp, dinv, b3, W3):
    return pl.pallas_call(
        _tc_last_body,
        grid=(_GRID,),
        in_specs=[
            pl.BlockSpec((2, _BLK, D_HID), lambda i: (0, i, 0)),
            pl.BlockSpec((_BLK, D_HID), lambda i: (i, 0)),
            pl.BlockSpec((_BLK, 1), lambda i: (i, 0)),
            pl.BlockSpec((1, N_CLASSES), lambda i: (0, 0)),
            pl.BlockSpec((D_HID, N_CLASSES), lambda i: (0, 0)),
        ],
        out_specs=pl.BlockSpec((_BLK, N_CLASSES), lambda i: (i, 0)),
        out_shape=jax.ShapeDtypeStruct((N, N_CLASSES), jnp.float32),
    )(a, hp, dinv, b3, W3)


@jax.jit
def kernel(x, edge_index, batch, W1, b1, W2, b2, W3, b3):
    per_tile = E // 32
    pad_pt = E_PAD // 32 - per_tile
    src2 = edge_index[0].reshape(32, per_tile)
    dst2 = edge_index[1].reshape(32, per_tile)
    pad_src = jnp.zeros((32, pad_pt), jnp.int32)
    pad_dst = jnp.broadcast_to(
        N + (jnp.arange(pad_pt, dtype=jnp.int32) % 16), (32, pad_pt))
    src_p = jnp.concatenate([src2, pad_src], axis=1).reshape(N_CHUNKS, CHUNK)
    dst_p = jnp.concatenate([dst2, pad_dst], axis=1).reshape(N_CHUNKS, CHUNK)

    deg2 = _deg_kernel(dst_p)
    deg3 = deg2[:, :N, None]

    hp1, dinv = _tc_first(x, deg3, W1)
    a1 = _agg_kernel(src_p, dst_p, hp1)
    hp2 = _tc_mid(a1, hp1, dinv, b1.reshape(1, -1), W2)
    a2 = _agg_kernel(src_p, dst_p, hp2)
    hpz = _tc_scale(a2, hp2, dinv, b2.reshape(1, -1))
    a3 = _agg_kernel(src_p, dst_p, hpz)
    return _tc_last(a3, hpz, dinv, b3.reshape(1, -1), W3)


# speedup vs baseline: 8.0115x; 1.0325x over previous
import functools

import jax
import jax.numpy as jnp
from jax import lax
from jax.experimental import pallas as pl
from jax.experimental.pallas import tpu as pltpu
from jax.experimental.pallas import tpu_sc as plsc

N = 10000
E = 320000
D_FEAT = 128
D_HID = 128
N_CLASSES = 64

CHUNK = 128
N_CHUNKS = 2560
E_PAD = N_CHUNKS * CHUNK
CPT = N_CHUNKS // 32
IBLK = 16
ROW_SPLIT = 624
ACC_ROWS = 10016
DEG_SLOTS = 10016

_MESH = plsc.VectorSubcoreMesh(
    core_axis_name="c", subcore_axis_name="s", num_cores=2, num_subcores=16)


def _deg_body(dst_hbm, out_hbm, idx_v, hist):
    c = lax.axis_index("c")
    s = lax.axis_index("s")
    wid = c * 16 + s
    zero16 = jnp.zeros((16,), jnp.float32)

    def zstep(i, _):
        hist[pl.ds(16 * i, 16)] = zero16
        return 0

    lax.fori_loop(0, DEG_SLOTS // 16, zstep, 0)
    row0 = wid * CPT
    pltpu.sync_copy(dst_hbm.at[pl.ds(row0, CPT), :], idx_v)

    ones16 = jnp.ones((16,), jnp.float32)

    def step(g, _):
        for j in range(0, CHUNK, 16):
            plsc.addupdate_scatter(hist, [idx_v[g, pl.ds(j, 16)]], ones16)
        return 0

    lax.fori_loop(0, CPT, step, 0)
    pltpu.sync_copy(hist, out_hbm.at[wid])


_deg_kernel = functools.partial(
    pl.kernel,
    out_type=jax.ShapeDtypeStruct((32, DEG_SLOTS), jnp.float32),
    mesh=_MESH,
    compiler_params=pltpu.CompilerParams(needs_layout_passes=False),
    scratch_types=[
        pltpu.VMEM((CPT, CHUNK), jnp.int32),
        pltpu.VMEM((DEG_SLOTS,), jnp.float32),
    ],
)(_deg_body)


def _agg_body(src_hbm, dst_hbm, hp, out_hbm,
              src_idx, dst_idx, rows, acc, g0, g1, s0, s1):
    c = lax.axis_index("c")
    s = lax.axis_index("s")
    off = s * ROW_SPLIT
    pl.when(s < 15)(lambda: pltpu.sync_copy(
        hp.at[pl.ds(off, ROW_SPLIT), :], acc.at[pl.ds(off, ROW_SPLIT), :]))
    pl.when(s == 15)(lambda: pltpu.sync_copy(
        hp.at[pl.ds(15 * ROW_SPLIT, 640), :],
        acc.at[pl.ds(15 * ROW_SPLIT, 640), :]))
    row0 = c * (16 * CPT) + s * CPT
    plsc.subcore_barrier()

    def gather(k, buf, sem):
        pltpu.async_copy(hp.at[src_idx.at[k]], rows.at[buf], sem)

    def gwait(k, buf, sem):
        pltpu.make_async_copy(hp.at[src_idx.at[k]], rows.at[buf], sem).wait()

    def scat(k, buf, sem):
        pltpu.async_copy(rows.at[buf], acc.at[dst_idx.at[k]], sem, add=True)

    def swait(k, buf, sem):
        pltpu.make_async_copy(rows.at[buf], acc.at[dst_idx.at[k]],
                              sem).wait()

    for blk in range(CPT // IBLK):
        pltpu.sync_copy(src_hbm.at[pl.ds(row0 + blk * IBLK, IBLK), :],
                        src_idx)
        pltpu.sync_copy(dst_hbm.at[pl.ds(row0 + blk * IBLK, IBLK), :],
                        dst_idx)
        gather(0, 0, g0)

        def step(i, _):
            k0 = 2 * i
            k1 = 2 * i + 1
            gwait(k0, 0, g0)
            pl.when(i > 0)(lambda: swait(k1 - 2, 1, s1))
            gather(k1, 1, g1)
            scat(k0, 0, s0)
            gwait(k1, 1, g1)
            pl.when(i < IBLK // 2 - 1)(lambda: swait(k0, 0, s0))
            pl.when(i < IBLK // 2 - 1)(lambda: gather(k0 + 2, 0, g0))
            scat(k1, 1, s1)
            return 0

        lax.fori_loop(0, IBLK // 2, step, 0)
        swait(IBLK - 2, 0, s0)
        swait(IBLK - 1, 1, s1)
    plsc.subcore_barrier()
    pl.when(s < 15)(lambda: pltpu.sync_copy(
        acc.at[pl.ds(off, ROW_SPLIT), :],
        out_hbm.at[c, pl.ds(off, ROW_SPLIT), :]))
    pl.when(s == 15)(lambda: pltpu.sync_copy(
        acc.at[pl.ds(15 * ROW_SPLIT, 640), :],
        out_hbm.at[c, pl.ds(15 * ROW_SPLIT, 640), :]))


_agg_kernel = functools.partial(
    pl.kernel,
    out_type=jax.ShapeDtypeStruct((2, N, D_HID), jnp.float32),
    mesh=_MESH,
    scratch_types=[
        pltpu.VMEM((IBLK, CHUNK), jnp.int32),
        pltpu.VMEM((IBLK, CHUNK), jnp.int32),
        pltpu.VMEM((2, CHUNK, D_HID), jnp.float32),
        pltpu.VMEM_SHARED((ACC_ROWS, D_HID), jnp.float32),
        pltpu.SemaphoreType.DMA,
        pltpu.SemaphoreType.DMA,
        pltpu.SemaphoreType.DMA,
        pltpu.SemaphoreType.DMA,
    ],
)(_agg_body)


_BLK = 400
_GRID = N // _BLK


def _tc_first_body(x_ref, deg_ref, w_ref, hp_ref, dinv_ref):
    deg = jnp.sum(deg_ref[...], axis=0) + 1.0
    dinv = lax.rsqrt(deg)
    h = jnp.dot(x_ref[...], w_ref[...], preferred_element_type=jnp.float32)
    hp_ref[...] = h * dinv
    dinv_ref[...] = dinv


def _tc_first(x, deg3, W1):
    return pl.pallas_call(
        _tc_first_body,
        grid=(_GRID,),
        in_specs=[
            pl.BlockSpec((_BLK, D_FEAT), lambda i: (i, 0)),
            pl.BlockSpec((32, _BLK, 1), lambda i: (0, i, 0)),
            pl.BlockSpec((D_FEAT, D_HID), lambda i: (0, 0)),
        ],
        out_specs=[
            pl.BlockSpec((_BLK, D_HID), lambda i: (i, 0)),
            pl.BlockSpec((_BLK, 1), lambda i: (i, 0)),
        ],
        out_shape=[
            jax.ShapeDtypeStruct((N, D_HID), jnp.float32),
            jax.ShapeDtypeStruct((N, 1), jnp.float32),
        ],
    )(x, deg3, W1)


def _tc_mid_body(a_ref, hp_ref, dinv_ref, b_ref, w_ref, out_ref):
    agg = a_ref[0] + a_ref[1] - hp_ref[...]
    dinv = dinv_ref[...]
    z = jnp.maximum(agg * dinv + b_ref[...], 0.0)
    h = jnp.dot(z, w_ref[...], preferred_element_type=jnp.float32)
    out_ref[...] = h * dinv


def _tc_mid(a, hp, dinv, b, W):
    return pl.pallas_call(
        _tc_mid_body,
        grid=(_GRID,),
        in_specs=[
            pl.BlockSpec((2, _BLK, D_HID), lambda i: (0, i, 0)),
            pl.BlockSpec((_BLK, D_HID), lambda i: (i, 0)),
            pl.BlockSpec((_BLK, 1), lambda i: (i, 0)),
            pl.BlockSpec((1, D_HID), lambda i: (0, 0)),
            pl.BlockSpec((D_HID, D_HID), lambda i: (0, 0)),
        ],
        out_specs=pl.BlockSpec((_BLK, D_HID), lambda i: (i, 0)),
        out_shape=jax.ShapeDtypeStruct((N, D_HID), jnp.float32),
    )(a, hp, dinv, b, W)


def _tc_scale_body(a_ref, hp_ref, dinv_ref, b_ref, out_ref):
    agg = a_ref[0] + a_ref[1] - hp_ref[...]
    dinv = dinv_ref[...]
    z = jnp.maximum(agg * dinv + b_ref[...], 0.0)
    out_ref[...] = z * dinv


def _tc_scale(a, hp, dinv, b):
    return pl.pallas_call(
        _tc_scale_body,
        grid=(_GRID,),
        in_specs=[
            pl.BlockSpec((2, _BLK, D_HID), lambda i: (0, i, 0)),
            pl.BlockSpec((_BLK, D_HID), lambda i: (i, 0)),
            pl.BlockSpec((_BLK, 1), lambda i: (i, 0)),
            pl.BlockSpec((1, D_HID), lambda i: (0, 0)),
        ],
        out_specs=pl.BlockSpec((_BLK, D_HID), lambda i: (i, 0)),
        out_shape=jax.ShapeDtypeStruct((N, D_HID), jnp.float32),
    )(a, hp, dinv, b)


def _tc_last_body(a_ref, hp_ref, dinv_ref, b_ref, w_ref, out_ref):
    agg = a_ref[0] + a_ref[1] - hp_ref[...]
    h = jnp.dot(agg, w_ref[...], preferred_element_type=jnp.float32)
    y = h * dinv_ref[...] + b_ref[...]
    m = jnp.max(y, axis=1, keepdims=True)
    lse = jnp.log(jnp.sum(jnp.exp(y - m), axis=1, keepdims=True))
    out_ref[...] = y - m - lse


def _tc_last(a, hp, dinv, b3, W3):
    return pl.pallas_call(
        _tc_last_body,
        grid=(_GRID,),
        in_specs=[
            pl.BlockSpec((2, _BLK, D_HID), lambda i: (0, i, 0)),
            pl.BlockSpec((_BLK, D_HID), lambda i: (i, 0)),
            pl.BlockSpec((_BLK, 1), lambda i: (i, 0)),
            pl.BlockSpec((1, N_CLASSES), lambda i: (0, 0)),
            pl.BlockSpec((D_HID, N_CLASSES), lambda i: (0, 0)),
        ],
        out_specs=pl.BlockSpec((_BLK, N_CLASSES), lambda i: (i, 0)),
        out_shape=jax.ShapeDtypeStruct((N, N_CLASSES), jnp.float32),
    )(a, hp, dinv, b3, W3)


@jax.jit
def kernel(x, edge_index, batch, W1, b1, W2, b2, W3, b3):
    per_tile = E // 32
    pad_pt = E_PAD // 32 - per_tile
    src2 = edge_index[0].reshape(32, per_tile)
    dst2 = edge_index[1].reshape(32, per_tile)
    pad_src = jnp.zeros((32, pad_pt), jnp.int32)
    pad_dst = jnp.broadcast_to(
        N + (jnp.arange(pad_pt, dtype=jnp.int32) % 16), (32, pad_pt))
    src_p = jnp.concatenate([src2, pad_src], axis=1).reshape(N_CHUNKS, CHUNK)
    dst_p = jnp.concatenate([dst2, pad_dst], axis=1).reshape(N_CHUNKS, CHUNK)

    deg2 = _deg_kernel(dst_p)
    deg3 = deg2[:, :N, None]

    hp1, dinv = _tc_first(x, deg3, W1)
    a1 = _agg_kernel(src_p, dst_p, hp1)
    hp2 = _tc_mid(a1, hp1, dinv, b1.reshape(1, -1), W2)
    a2 = _agg_kernel(src_p, dst_p, hp2)
    hpz = _tc_scale(a2, hp2, dinv, b2.reshape(1, -1))
    a3 = _agg_kernel(src_p, dst_p, hpz)
    return _tc_last(a3, hpz, dinv, b3.reshape(1, -1), W3)


# speedup vs baseline: 8.0468x; 1.0044x over previous
import functools

import jax
import jax.numpy as jnp
from jax import lax
from jax.experimental import pallas as pl
from jax.experimental.pallas import tpu as pltpu
from jax.experimental.pallas import tpu_sc as plsc

N = 10000
E = 320000
D_FEAT = 128
D_HID = 128
N_CLASSES = 64

CHUNK = 128
N_CHUNKS = 2560
E_PAD = N_CHUNKS * CHUNK
CPT = N_CHUNKS // 32
IBLK = 16
ROW_SPLIT = 624
ACC_ROWS = 10016
DEG_SLOTS = 10016

_MESH = plsc.VectorSubcoreMesh(
    core_axis_name="c", subcore_axis_name="s", num_cores=2, num_subcores=16)


def _deg_body(dst_hbm, out_hbm, idx_v, hist):
    c = lax.axis_index("c")
    s = lax.axis_index("s")
    wid = c * 16 + s
    zero16 = jnp.zeros((16,), jnp.float32)

    def zstep(i, _):
        hist[pl.ds(16 * i, 16)] = zero16
        return 0

    lax.fori_loop(0, DEG_SLOTS // 16, zstep, 0)
    row0 = wid * CPT
    pltpu.sync_copy(dst_hbm.at[pl.ds(row0, CPT), :], idx_v)

    ones16 = jnp.ones((16,), jnp.float32)

    def step(g, _):
        for j in range(0, CHUNK, 16):
            plsc.addupdate_scatter(hist, [idx_v[g, pl.ds(j, 16)]], ones16)
        return 0

    lax.fori_loop(0, CPT, step, 0)
    pltpu.sync_copy(hist, out_hbm.at[wid])


_deg_kernel = functools.partial(
    pl.kernel,
    out_type=jax.ShapeDtypeStruct((32, DEG_SLOTS), jnp.float32),
    mesh=_MESH,
    compiler_params=pltpu.CompilerParams(needs_layout_passes=False),
    scratch_types=[
        pltpu.VMEM((CPT, CHUNK), jnp.int32),
        pltpu.VMEM((DEG_SLOTS,), jnp.float32),
    ],
)(_deg_body)


def _agg_body(src_hbm, dst_hbm, hp, out_hbm,
              src_idx, dst_idx, rows, acc, g0, g1, g2, g3, s0, s1):
    gsemA = (g0, g1)
    gsemB = (g2, g3)
    c = lax.axis_index("c")
    s = lax.axis_index("s")
    off = s * ROW_SPLIT
    pl.when(s < 15)(lambda: pltpu.sync_copy(
        hp.at[pl.ds(off, ROW_SPLIT), :], acc.at[pl.ds(off, ROW_SPLIT), :]))
    pl.when(s == 15)(lambda: pltpu.sync_copy(
        hp.at[pl.ds(15 * ROW_SPLIT, 640), :],
        acc.at[pl.ds(15 * ROW_SPLIT, 640), :]))
    row0 = c * (16 * CPT) + s * CPT
    plsc.subcore_barrier()

    def gather(k, buf, sem):
        pltpu.async_copy(hp.at[src_idx.at[k, pl.ds(0, 64)]],
                         rows.at[buf, pl.ds(0, 64), :], gsemA[buf])
        pltpu.async_copy(hp.at[src_idx.at[k, pl.ds(64, 64)]],
                         rows.at[buf, pl.ds(64, 64), :], gsemB[buf])

    def gwait(k, buf, sem):
        pltpu.make_async_copy(hp.at[src_idx.at[k, pl.ds(0, 64)]],
                              rows.at[buf, pl.ds(0, 64), :], gsemA[buf]).wait()
        pltpu.make_async_copy(hp.at[src_idx.at[k, pl.ds(64, 64)]],
                              rows.at[buf, pl.ds(64, 64), :], gsemB[buf]).wait()

    def scat(k, buf, sem):
        pltpu.async_copy(rows.at[buf], acc.at[dst_idx.at[k]], sem, add=True)

    def swait(k, buf, sem):
        pltpu.make_async_copy(rows.at[buf], acc.at[dst_idx.at[k]],
                              sem).wait()

    for blk in range(CPT // IBLK):
        pltpu.sync_copy(src_hbm.at[pl.ds(row0 + blk * IBLK, IBLK), :],
                        src_idx)
        pltpu.sync_copy(dst_hbm.at[pl.ds(row0 + blk * IBLK, IBLK), :],
                        dst_idx)
        gather(0, 0, g0)

        def step(i, _):
            k0 = 2 * i
            k1 = 2 * i + 1
            gwait(k0, 0, g0)
            pl.when(i > 0)(lambda: swait(k1 - 2, 1, s1))
            gather(k1, 1, g1)
            scat(k0, 0, s0)
            gwait(k1, 1, g1)
            pl.when(i < IBLK // 2 - 1)(lambda: swait(k0, 0, s0))
            pl.when(i < IBLK // 2 - 1)(lambda: gather(k0 + 2, 0, g0))
            scat(k1, 1, s1)
            return 0

        lax.fori_loop(0, IBLK // 2, step, 0)
        swait(IBLK - 2, 0, s0)
        swait(IBLK - 1, 1, s1)
    plsc.subcore_barrier()
    pl.when(s < 15)(lambda: pltpu.sync_copy(
        acc.at[pl.ds(off, ROW_SPLIT), :],
        out_hbm.at[c, pl.ds(off, ROW_SPLIT), :]))
    pl.when(s == 15)(lambda: pltpu.sync_copy(
        acc.at[pl.ds(15 * ROW_SPLIT, 640), :],
        out_hbm.at[c, pl.ds(15 * ROW_SPLIT, 640), :]))


_agg_kernel = functools.partial(
    pl.kernel,
    out_type=jax.ShapeDtypeStruct((2, N, D_HID), jnp.float32),
    mesh=_MESH,
    scratch_types=[
        pltpu.VMEM((IBLK, CHUNK), jnp.int32),
        pltpu.VMEM((IBLK, CHUNK), jnp.int32),
        pltpu.VMEM((2, CHUNK, D_HID), jnp.float32),
        pltpu.VMEM_SHARED((ACC_ROWS, D_HID), jnp.float32),
        pltpu.SemaphoreType.DMA,
        pltpu.SemaphoreType.DMA,
        pltpu.SemaphoreType.DMA,
        pltpu.SemaphoreType.DMA,
        pltpu.SemaphoreType.DMA,
        pltpu.SemaphoreType.DMA,
    ],
)(_agg_body)


_BLK = 400
_GRID = N // _BLK


def _tc_first_body(x_ref, deg_ref, w_ref, hp_ref, dinv_ref):
    deg = jnp.sum(deg_ref[...], axis=0) + 1.0
    dinv = lax.rsqrt(deg)
    h = jnp.dot(x_ref[...], w_ref[...], preferred_element_type=jnp.float32)
    hp_ref[...] = h * dinv
    dinv_ref[...] = dinv


def _tc_first(x, deg3, W1):
    return pl.pallas_call(
        _tc_first_body,
        grid=(_GRID,),
        in_specs=[
            pl.BlockSpec((_BLK, D_FEAT), lambda i: (i, 0)),
            pl.BlockSpec((32, _BLK, 1), lambda i: (0, i, 0)),
            pl.BlockSpec((D_FEAT, D_HID), lambda i: (0, 0)),
        ],
        out_specs=[
            pl.BlockSpec((_BLK, D_HID), lambda i: (i, 0)),
            pl.BlockSpec((_BLK, 1), lambda i: (i, 0)),
        ],
        out_shape=[
            jax.ShapeDtypeStruct((N, D_HID), jnp.float32),
            jax.ShapeDtypeStruct((N, 1), jnp.float32),
        ],
    )(x, deg3, W1)


def _tc_mid_body(a_ref, hp_ref, dinv_ref, b_ref, w_ref, out_ref):
    agg = a_ref[0] + a_ref[1] - hp_ref[...]
    dinv = dinv_ref[...]
    z = jnp.maximum(agg * dinv + b_ref[...], 0.0)
    h = jnp.dot(z, w_ref[...], preferred_element_type=jnp.float32)
    out_ref[...] = h * dinv


def _tc_mid(a, hp, dinv, b, W):
    return pl.pallas_call(
        _tc_mid_body,
        grid=(_GRID,),
        in_specs=[
            pl.BlockSpec((2, _BLK, D_HID), lambda i: (0, i, 0)),
            pl.BlockSpec((_BLK, D_HID), lambda i: (i, 0)),
            pl.BlockSpec((_BLK, 1), lambda i: (i, 0)),
            pl.BlockSpec((1, D_HID), lambda i: (0, 0)),
            pl.BlockSpec((D_HID, D_HID), lambda i: (0, 0)),
        ],
        out_specs=pl.BlockSpec((_BLK, D_HID), lambda i: (i, 0)),
        out_shape=jax.ShapeDtypeStruct((N, D_HID), jnp.float32),
    )(a, hp, dinv, b, W)


def _tc_scale_body(a_ref, hp_ref, dinv_ref, b_ref, out_ref):
    agg = a_ref[0] + a_ref[1] - hp_ref[...]
    dinv = dinv_ref[...]
    z = jnp.maximum(agg * dinv + b_ref[...], 0.0)
    out_ref[...] = z * dinv


def _tc_scale(a, hp, dinv, b):
    return pl.pallas_call(
        _tc_scale_body,
        grid=(_GRID,),
        in_specs=[
            pl.BlockSpec((2, _BLK, D_HID), lambda i: (0, i, 0)),
            pl.BlockSpec((_BLK, D_HID), lambda i: (i, 0)),
            pl.BlockSpec((_BLK, 1), lambda i: (i, 0)),
            pl.BlockSpec((1, D_HID), lambda i: (0, 0)),
        ],
        out_specs=pl.BlockSpec((_BLK, D_HID), lambda i: (i, 0)),
        out_shape=jax.ShapeDtypeStruct((N, D_HID), jnp.float32),
    )(a, hp, dinv, b)


def _tc_last_body(a_ref, hp_ref, dinv_ref, b_ref, w_ref, out_ref):
    agg = a_ref[0] + a_ref[1] - hp_ref[...]
    h = jnp.dot(agg, w_ref[...], preferred_element_type=jnp.float32)
    y = h * dinv_ref[...] + b_ref[...]
    m = jnp.max(y, axis=1, keepdims=True)
    lse = jnp.log(jnp.sum(jnp.exp(y - m), axis=1, keepdims=True))
    out_ref[...] = y - m - lse


def _tc_last(a, hp, dinv, b3, W3):
    return pl.pallas_call(
        _tc_last_body,
        grid=(_GRID,),
        in_specs=[
            pl.BlockSpec((2, _BLK, D_HID), lambda i: (0, i, 0)),
            pl.BlockSpec((_BLK, D_HID), lambda i: (i, 0)),
            pl.BlockSpec((_BLK, 1), lambda i: (i, 0)),
            pl.BlockSpec((1, N_CLASSES), lambda i: (0, 0)),
            pl.BlockSpec((D_HID, N_CLASSES), lambda i: (0, 0)),
        ],
        out_specs=pl.BlockSpec((_BLK, N_CLASSES), lambda i: (i, 0)),
        out_shape=jax.ShapeDtypeStruct((N, N_CLASSES), jnp.float32),
    )(a, hp, dinv, b3, W3)


@jax.jit
def kernel(x, edge_index, batch, W1, b1, W2, b2, W3, b3):
    per_tile = E // 32
    pad_pt = E_PAD // 32 - per_tile
    src2 = edge_index[0].reshape(32, per_tile)
    dst2 = edge_index[1].reshape(32, per_tile)
    pad_src = jnp.zeros((32, pad_pt), jnp.int32)
    pad_dst = jnp.broadcast_to(
        N + (jnp.arange(pad_pt, dtype=jnp.int32) % 16), (32, pad_pt))
    src_p = jnp.concatenate([src2, pad_src], axis=1).reshape(N_CHUNKS, CHUNK)
    dst_p = jnp.concatenate([dst2, pad_dst], axis=1).reshape(N_CHUNKS, CHUNK)

    deg2 = _deg_kernel(dst_p)
    deg3 = deg2[:, :N, None]

    hp1, dinv = _tc_first(x, deg3, W1)
    a1 = _agg_kernel(src_p, dst_p, hp1)
    hp2 = _tc_mid(a1, hp1, dinv, b1.reshape(1, -1), W2)
    a2 = _agg_kernel(src_p, dst_p, hp2)
    hpz = _tc_scale(a2, hp2, dinv, b2.reshape(1, -1))
    a3 = _agg_kernel(src_p, dst_p, hpz)
    return _tc_last(a3, hpz, dinv, b3.reshape(1, -1), W3)


# speedup vs baseline: 8.1279x; 1.0101x over previous
import functools

import jax
import jax.numpy as jnp
from jax import lax
from jax.experimental import pallas as pl
from jax.experimental.pallas import tpu as pltpu
from jax.experimental.pallas import tpu_sc as plsc

N = 10000
E = 320000
D_FEAT = 128
D_HID = 128
N_CLASSES = 64

CHUNK = 128
N_CHUNKS = 2560
E_PAD = N_CHUNKS * CHUNK
CPT = N_CHUNKS // 32
IBLK = 40
ROW_SPLIT = 624
ACC_ROWS = 10016
DEG_SLOTS = 10016

_MESH = plsc.VectorSubcoreMesh(
    core_axis_name="c", subcore_axis_name="s", num_cores=2, num_subcores=16)


def _deg_body(dst_hbm, out_hbm, idx_v, hist):
    c = lax.axis_index("c")
    s = lax.axis_index("s")
    wid = c * 16 + s
    zero16 = jnp.zeros((16,), jnp.float32)

    def zstep(i, _):
        hist[pl.ds(16 * i, 16)] = zero16
        return 0

    lax.fori_loop(0, DEG_SLOTS // 16, zstep, 0)
    row0 = wid * CPT
    pltpu.sync_copy(dst_hbm.at[pl.ds(row0, CPT), :], idx_v)

    ones16 = jnp.ones((16,), jnp.float32)

    def step(g, _):
        for j in range(0, CHUNK, 16):
            plsc.addupdate_scatter(hist, [idx_v[g, pl.ds(j, 16)]], ones16)
        return 0

    lax.fori_loop(0, CPT, step, 0)
    pltpu.sync_copy(hist, out_hbm.at[wid])


_deg_kernel = functools.partial(
    pl.kernel,
    out_type=jax.ShapeDtypeStruct((32, DEG_SLOTS), jnp.float32),
    mesh=_MESH,
    compiler_params=pltpu.CompilerParams(needs_layout_passes=False),
    scratch_types=[
        pltpu.VMEM((CPT, CHUNK), jnp.int32),
        pltpu.VMEM((DEG_SLOTS,), jnp.float32),
    ],
)(_deg_body)


def _agg_body(src_hbm, dst_hbm, hp, out_hbm,
              src_idx, dst_idx, rows, acc, g0, g1, s0, s1):
    c = lax.axis_index("c")
    s = lax.axis_index("s")
    off = s * ROW_SPLIT
    pl.when(s < 15)(lambda: pltpu.sync_copy(
        hp.at[pl.ds(off, ROW_SPLIT), :], acc.at[pl.ds(off, ROW_SPLIT), :]))
    pl.when(s == 15)(lambda: pltpu.sync_copy(
        hp.at[pl.ds(15 * ROW_SPLIT, 640), :],
        acc.at[pl.ds(15 * ROW_SPLIT, 640), :]))
    row0 = c * (16 * CPT) + s * CPT
    plsc.subcore_barrier()

    def gather(k, buf, sem):
        pltpu.async_copy(hp.at[src_idx.at[k]], rows.at[buf], sem)

    def gwait(k, buf, sem):
        pltpu.make_async_copy(hp.at[src_idx.at[k]], rows.at[buf], sem).wait()

    def scat(k, buf, sem):
        pltpu.async_copy(rows.at[buf], acc.at[dst_idx.at[k]], sem, add=True)

    def swait(k, buf, sem):
        pltpu.make_async_copy(rows.at[buf], acc.at[dst_idx.at[k]],
                              sem).wait()

    for blk in range(CPT // IBLK):
        pltpu.sync_copy(src_hbm.at[pl.ds(row0 + blk * IBLK, IBLK), :],
                        src_idx)
        pltpu.sync_copy(dst_hbm.at[pl.ds(row0 + blk * IBLK, IBLK), :],
                        dst_idx)
        gather(0, 0, g0)

        def step(i, _):
            k0 = 2 * i
            k1 = 2 * i + 1
            gwait(k0, 0, g0)
            pl.when(i > 0)(lambda: swait(k1 - 2, 1, s1))
            gather(k1, 1, g1)
            scat(k0, 0, s0)
            gwait(k1, 1, g1)
            pl.when(i < IBLK // 2 - 1)(lambda: swait(k0, 0, s0))
            pl.when(i < IBLK // 2 - 1)(lambda: gather(k0 + 2, 0, g0))
            scat(k1, 1, s1)
            return 0

        lax.fori_loop(0, IBLK // 2, step, 0)
        swait(IBLK - 2, 0, s0)
        swait(IBLK - 1, 1, s1)
    plsc.subcore_barrier()
    pl.when(s < 15)(lambda: pltpu.sync_copy(
        acc.at[pl.ds(off, ROW_SPLIT), :],
        out_hbm.at[c, pl.ds(off, ROW_SPLIT), :]))
    pl.when(s == 15)(lambda: pltpu.sync_copy(
        acc.at[pl.ds(15 * ROW_SPLIT, 640), :],
        out_hbm.at[c, pl.ds(15 * ROW_SPLIT, 640), :]))


_agg_kernel = functools.partial(
    pl.kernel,
    out_type=jax.ShapeDtypeStruct((2, N, D_HID), jnp.float32),
    mesh=_MESH,
    scratch_types=[
        pltpu.VMEM((IBLK, CHUNK), jnp.int32),
        pltpu.VMEM((IBLK, CHUNK), jnp.int32),
        pltpu.VMEM((2, CHUNK, D_HID), jnp.float32),
        pltpu.VMEM_SHARED((ACC_ROWS, D_HID), jnp.float32),
        pltpu.SemaphoreType.DMA,
        pltpu.SemaphoreType.DMA,
        pltpu.SemaphoreType.DMA,
        pltpu.SemaphoreType.DMA,
    ],
)(_agg_body)


_BLK = 400
_GRID = N // _BLK


def _tc_first_body(x_ref, deg_ref, w_ref, hp_ref, dinv_ref):
    deg = jnp.sum(deg_ref[...], axis=0) + 1.0
    dinv = lax.rsqrt(deg)
    h = jnp.dot(x_ref[...], w_ref[...], preferred_element_type=jnp.float32)
    hp_ref[...] = h * dinv
    dinv_ref[...] = dinv


def _tc_first(x, deg3, W1):
    return pl.pallas_call(
        _tc_first_body,
        grid=(_GRID,),
        in_specs=[
            pl.BlockSpec((_BLK, D_FEAT), lambda i: (i, 0)),
            pl.BlockSpec((32, _BLK, 1), lambda i: (0, i, 0)),
            pl.BlockSpec((D_FEAT, D_HID), lambda i: (0, 0)),
        ],
        out_specs=[
            pl.BlockSpec((_BLK, D_HID), lambda i: (i, 0)),
            pl.BlockSpec((_BLK, 1), lambda i: (i, 0)),
        ],
        out_shape=[
            jax.ShapeDtypeStruct((N, D_HID), jnp.float32),
            jax.ShapeDtypeStruct((N, 1), jnp.float32),
        ],
    )(x, deg3, W1)


def _tc_mid_body(a_ref, hp_ref, dinv_ref, b_ref, w_ref, out_ref):
    agg = a_ref[0] + a_ref[1] - hp_ref[...]
    dinv = dinv_ref[...]
    z = jnp.maximum(agg * dinv + b_ref[...], 0.0)
    h = jnp.dot(z, w_ref[...], preferred_element_type=jnp.float32)
    out_ref[...] = h * dinv


def _tc_mid(a, hp, dinv, b, W):
    return pl.pallas_call(
        _tc_mid_body,
        grid=(_GRID,),
        in_specs=[
            pl.BlockSpec((2, _BLK, D_HID), lambda i: (0, i, 0)),
            pl.BlockSpec((_BLK, D_HID), lambda i: (i, 0)),
            pl.BlockSpec((_BLK, 1), lambda i: (i, 0)),
            pl.BlockSpec((1, D_HID), lambda i: (0, 0)),
            pl.BlockSpec((D_HID, D_HID), lambda i: (0, 0)),
        ],
        out_specs=pl.BlockSpec((_BLK, D_HID), lambda i: (i, 0)),
        out_shape=jax.ShapeDtypeStruct((N, D_HID), jnp.float32),
    )(a, hp, dinv, b, W)


def _tc_scale_body(a_ref, hp_ref, dinv_ref, b_ref, out_ref):
    agg = a_ref[0] + a_ref[1] - hp_ref[...]
    dinv = dinv_ref[...]
    z = jnp.maximum(agg * dinv + b_ref[...], 0.0)
    out_ref[...] = z * dinv


def _tc_scale(a, hp, dinv, b):
    return pl.pallas_call(
        _tc_scale_body,
        grid=(_GRID,),
        in_specs=[
            pl.BlockSpec((2, _BLK, D_HID), lambda i: (0, i, 0)),
            pl.BlockSpec((_BLK, D_HID), lambda i: (i, 0)),
            pl.BlockSpec((_BLK, 1), lambda i: (i, 0)),
            pl.BlockSpec((1, D_HID), lambda i: (0, 0)),
        ],
        out_specs=pl.BlockSpec((_BLK, D_HID), lambda i: (i, 0)),
        out_shape=jax.ShapeDtypeStruct((N, D_HID), jnp.float32),
    )(a, hp, dinv, b)


def _tc_last_body(a_ref, hp_ref, dinv_ref, b_ref, w_ref, out_ref):
    agg = a_ref[0] + a_ref[1] - hp_ref[...]
    h = jnp.dot(agg, w_ref[...], preferred_element_type=jnp.float32)
    y = h * dinv_ref[...] + b_ref[...]
    m = jnp.max(y, axis=1, keepdims=True)
    lse = jnp.log(jnp.sum(jnp.exp(y - m), axis=1, keepdims=True))
    out_ref[...] = y - m - lse


def _tc_last(a, hp, dinv, b3, W3):
    return pl.pallas_call(
        _tc_last_body,
        grid=(_GRID,),
        in_specs=[
            pl.BlockSpec((2, _BLK, D_HID), lambda i: (0, i, 0)),
            pl.BlockSpec((_BLK, D_HID), lambda i: (i, 0)),
            pl.BlockSpec((_BLK, 1), lambda i: (i, 0)),
            pl.BlockSpec((1, N_CLASSES), lambda i: (0, 0)),
            pl.BlockSpec((D_HID, N_CLASSES), lambda i: (0, 0)),
        ],
        out_specs=pl.BlockSpec((_BLK, N_CLASSES), lambda i: (i, 0)),
        out_shape=jax.ShapeDtypeStruct((N, N_CLASSES), jnp.float32),
    )(a, hp, dinv, b3, W3)


@jax.jit
def kernel(x, edge_index, batch, W1, b1, W2, b2, W3, b3):
    per_tile = E // 32
    pad_pt = E_PAD // 32 - per_tile
    src2 = edge_index[0].reshape(32, per_tile)
    dst2 = edge_index[1].reshape(32, per_tile)
    pad_src = jnp.zeros((32, pad_pt), jnp.int32)
    pad_dst = jnp.broadcast_to(
        N + (jnp.arange(pad_pt, dtype=jnp.int32) % 16), (32, pad_pt))
    src_p = jnp.concatenate([src2, pad_src], axis=1).reshape(N_CHUNKS, CHUNK)
    dst_p = jnp.concatenate([dst2, pad_dst], axis=1).reshape(N_CHUNKS, CHUNK)

    deg2 = _deg_kernel(dst_p)
    deg3 = deg2[:, :N, None]

    hp1, dinv = _tc_first(x, deg3, W1)
    a1 = _agg_kernel(src_p, dst_p, hp1)
    hp2 = _tc_mid(a1, hp1, dinv, b1.reshape(1, -1), W2)
    a2 = _agg_kernel(src_p, dst_p, hp2)
    hpz = _tc_scale(a2, hp2, dinv, b2.reshape(1, -1))
    a3 = _agg_kernel(src_p, dst_p, hpz)
    return _tc_last(a3, hpz, dinv, b3.reshape(1, -1), W3)
